# Initial kernel scaffold; baseline (speedup 1.0000x reference)
import jax
import jax.numpy as jnp
from jax.experimental import pallas as pl


def kernel(pos_enc, position_ids):
    raise NotImplementedError("write your pallas kernel here")


# speedup vs baseline: 2.1227x; 2.1227x over previous
import functools

import jax
import jax.numpy as jnp
from jax import lax
from jax.experimental import pallas as pl
from jax.experimental.pallas import tpu as pltpu, tpu_sc as plsc

HIDDEN = 1024
N_IDX = 4 * 8192

_info = plsc.get_sparse_core_info()
NC, NS = _info.num_cores, _info.num_subcores
NW = NC * NS
B_PER_W = N_IDX // NW
CHUNK = 64
N_CHUNKS = B_PER_W // CHUNK


def _gather_body(table_hbm, idx_hbm, out_hbm, idx_v, rows_v, sem):
    wid = lax.axis_index("s") * NC + lax.axis_index("c")
    base = wid * B_PER_W
    pltpu.sync_copy(idx_hbm.at[pl.ds(base, B_PER_W)], idx_v)
    for c in range(N_CHUNKS):
        pltpu.async_copy(
            table_hbm.at[idx_v.at[pl.ds(c * CHUNK, CHUNK)]], rows_v, sem
        ).wait()
        pltpu.sync_copy(rows_v, out_hbm.at[pl.ds(base + c * CHUNK, CHUNK)])


_mesh = plsc.VectorSubcoreMesh(core_axis_name="c", subcore_axis_name="s")

_gather = pl.kernel(
    _gather_body,
    mesh=_mesh,
    out_type=jax.ShapeDtypeStruct((N_IDX, HIDDEN), jnp.float32),
    scratch_types=[
        pltpu.VMEM((B_PER_W,), jnp.int32),
        pltpu.VMEM((CHUNK, HIDDEN), jnp.float32),
        pltpu.SemaphoreType.DMA,
    ],
)


def kernel(pos_enc, position_ids):
    b, s = position_ids.shape
    idx = position_ids.reshape(-1).astype(jnp.int32)
    out = _gather(pos_enc, idx)
    return out.reshape(b, s, pos_enc.shape[1])


# speedup vs baseline: 2.2469x; 1.0585x over previous
import jax
import jax.numpy as jnp
from jax import lax
from jax.experimental import pallas as pl
from jax.experimental.pallas import tpu as pltpu, tpu_sc as plsc

HIDDEN = 1024
N_IDX = 4 * 8192

_info = plsc.get_sparse_core_info()
NC, NS = _info.num_cores, _info.num_subcores
NW = NC * NS
B_PER_W = N_IDX // NW
CHUNK = 32
NBUF = 2
N_CHUNKS = B_PER_W // CHUNK
ROUNDS = N_CHUNKS // NBUF


def _gather_body(table_hbm, idx_hbm, out_hbm,
                 idx_v, rows0, rows1, gsem0, gsem1, osem0, osem1):
    rows = (rows0, rows1)
    gsem = (gsem0, gsem1)
    osem = (osem0, osem1)
    wid = lax.axis_index("s") * NC + lax.axis_index("c")
    base = wid * B_PER_W
    pltpu.sync_copy(idx_hbm.at[pl.ds(base, B_PER_W)], idx_v)

    def gather(g, b):
        return pltpu.make_async_copy(
            table_hbm.at[idx_v.at[pl.ds(g * CHUNK, CHUNK)]], rows[b], gsem[b])

    def put(g, b):
        return pltpu.make_async_copy(
            rows[b], out_hbm.at[pl.ds(base + g * CHUNK, CHUNK)], osem[b])

    for b in range(NBUF):
        gather(b, b).start()

    def round_body(r, _):
        for b in range(NBUF):
            g = r * NBUF + b
            gather(g, b).wait()
            put(g, b).start()
        for b in range(NBUF):
            g = r * NBUF + b
            put(g, b).wait()
            gather(g + NBUF, b).start()
        return _

    lax.fori_loop(0, ROUNDS - 1, round_body, None)

    for b in range(NBUF):
        g = (ROUNDS - 1) * NBUF + b
        gather(g, b).wait()
        put(g, b).start()
    for b in range(NBUF):
        put((ROUNDS - 1) * NBUF + b, b).wait()


_mesh = plsc.VectorSubcoreMesh(core_axis_name="c", subcore_axis_name="s")

_gather = pl.kernel(
    _gather_body,
    mesh=_mesh,
    out_type=jax.ShapeDtypeStruct((N_IDX, HIDDEN), jnp.float32),
    scratch_types=[
        pltpu.VMEM((B_PER_W,), jnp.int32),
        pltpu.VMEM((CHUNK, HIDDEN), jnp.float32),
        pltpu.VMEM((CHUNK, HIDDEN), jnp.float32),
        pltpu.SemaphoreType.DMA,
        pltpu.SemaphoreType.DMA,
        pltpu.SemaphoreType.DMA,
        pltpu.SemaphoreType.DMA,
    ],
)


def kernel(pos_enc, position_ids):
    b, s = position_ids.shape
    idx = position_ids.reshape(-1).astype(jnp.int32)
    out = _gather(pos_enc, idx)
    return out.reshape(b, s, pos_enc.shape[1])


# speedup vs baseline: 2.3272x; 1.0357x over previous
import jax
import jax.numpy as jnp
from jax import lax
from jax.experimental import pallas as pl
from jax.experimental.pallas import tpu as pltpu, tpu_sc as plsc

HIDDEN = 1024
N_IDX = 4 * 8192

_info = plsc.get_sparse_core_info()
NC, NS = _info.num_cores, _info.num_subcores
NW = NC * NS
B_PER_W = N_IDX // NW
CHUNK = 16
NBUF = 4
N_CHUNKS = B_PER_W // CHUNK
ROUNDS = N_CHUNKS // NBUF


def _gather_body(table_hbm, idx_hbm, out_hbm, idx_v, rows_v,
                 gsem0, gsem1, gsem2, gsem3, osem0, osem1, osem2, osem3):
    gsem = (gsem0, gsem1, gsem2, gsem3)
    osem = (osem0, osem1, osem2, osem3)
    wid = lax.axis_index("s") * NC + lax.axis_index("c")
    base = wid * B_PER_W
    pltpu.sync_copy(idx_hbm.at[pl.ds(base, B_PER_W)], idx_v)

    def gather(g, b):
        return pltpu.make_async_copy(
            table_hbm.at[idx_v.at[pl.ds(g * CHUNK, CHUNK)]],
            rows_v.at[b], gsem[b])

    def put(g, b):
        return pltpu.make_async_copy(
            rows_v.at[b], out_hbm.at[pl.ds(base + g * CHUNK, CHUNK)], osem[b])

    for b in range(NBUF):
        gather(b, b).start()

    def round_body(r, _):
        for b in range(NBUF):
            g = r * NBUF + b
            gather(g, b).wait()
            put(g, b).start()
        for b in range(NBUF):
            g = r * NBUF + b
            put(g, b).wait()
            gather(g + NBUF, b).start()
        return _

    lax.fori_loop(0, ROUNDS - 1, round_body, None)

    for b in range(NBUF):
        g = (ROUNDS - 1) * NBUF + b
        gather(g, b).wait()
        put(g, b).start()
    for b in range(NBUF):
        put((ROUNDS - 1) * NBUF + b, b).wait()


_mesh = plsc.VectorSubcoreMesh(core_axis_name="c", subcore_axis_name="s")

_gather = pl.kernel(
    _gather_body,
    mesh=_mesh,
    out_type=jax.ShapeDtypeStruct((N_IDX, HIDDEN), jnp.float32),
    scratch_types=[
        pltpu.VMEM((B_PER_W,), jnp.int32),
        pltpu.VMEM((NBUF, CHUNK, HIDDEN), jnp.float32),
        pltpu.SemaphoreType.DMA,
        pltpu.SemaphoreType.DMA,
        pltpu.SemaphoreType.DMA,
        pltpu.SemaphoreType.DMA,
        pltpu.SemaphoreType.DMA,
        pltpu.SemaphoreType.DMA,
        pltpu.SemaphoreType.DMA,
        pltpu.SemaphoreType.DMA,
    ],
)


def kernel(pos_enc, position_ids):
    b, s = position_ids.shape
    idx = position_ids.reshape(-1).astype(jnp.int32)
    out = _gather(pos_enc, idx)
    return out.reshape(b, s, pos_enc.shape[1])
